# baseline (speedup 1.0000x reference)
import jax
import jax.numpy as jnp
from jax import lax
from jax.experimental import pallas as pl
from jax.experimental.pallas import tpu as pltpu
from jax.experimental.pallas import tpu_sc as plsc

EPS = 1e-05

B, C, H, W = 4, 96, 224, 224
HW = H * W
BLK = 512
NBLK = HW // BLK

NSUB = 16
CCH = 32
NCH = C // CCH
PPT = HW // NSUB
CHUNK = 128
NCHUNKS = HW // CHUNK
KMAX = (NCHUNKS + NSUB - 1) // NSUB
ZROWS = 196


def _proj_body(x_ref, w_ref, b_ref, xwt_ref, att_ref, dest_ref, off_ref,
               dst_ref):
    j = pl.program_id(1)
    xblk = x_ref[0]
    oa = jnp.dot(w_ref[...], xblk, preferred_element_type=jnp.float32)
    oa = oa + b_ref[...]
    off_y = oa[0:1] * float(H)
    off_x = oa[1:2] * float(W)
    att = jnp.exp(oa[2:3])

    p = j * BLK + lax.broadcasted_iota(jnp.int32, (1, BLK), 1)
    gy = (p // W).astype(jnp.float32)
    gx = (p - (p // W) * W).astype(jnp.float32)
    dy = jnp.round(gy + off_y).astype(jnp.int32)
    dx = jnp.round(gx + off_x).astype(jnp.int32)
    cy = jnp.clip(dy, 0, H - 1)
    cx = jnp.clip(dx, 0, W - 1)
    dest_ref[0, 0] = (cy * W + cx)[0]

    off_ref[0, 0] = off_y[0]
    off_ref[0, 1] = off_x[0]
    dst_ref[0, 0] = dy[0]
    dst_ref[0, 1] = dx[0]
    att_ref[0, 0] = att[0]
    for c3 in range(NCH):
        xwt_ref[c3] = (xblk[c3 * CCH:(c3 + 1) * CCH] * att).T


def _project(x_flat, Wc8, bc8):
    return pl.pallas_call(
        _proj_body,
        grid=(B, NBLK),
        in_specs=[
            pl.BlockSpec((1, C, BLK), lambda b, j: (b, 0, j)),
            pl.BlockSpec((8, C), lambda b, j: (0, 0)),
            pl.BlockSpec((8, 1), lambda b, j: (0, 0)),
        ],
        out_specs=[
            pl.BlockSpec((NCH, BLK, CCH), lambda b, j: (b, j, 0)),
            pl.BlockSpec((1, 1, BLK), lambda b, j: (b, 0, j)),
            pl.BlockSpec((1, 1, BLK), lambda b, j: (b, 0, j)),
            pl.BlockSpec((1, 2, BLK), lambda b, j: (b, 0, j)),
            pl.BlockSpec((1, 2, BLK), lambda b, j: (b, 0, j)),
        ],
        out_shape=[
            jax.ShapeDtypeStruct((B * NCH, HW, CCH), jnp.float32),
            jax.ShapeDtypeStruct((B, 1, HW), jnp.float32),
            jax.ShapeDtypeStruct((B, 1, HW), jnp.int32),
            jax.ShapeDtypeStruct((B, 2, HW), jnp.float32),
            jax.ShapeDtypeStruct((B, 2, HW), jnp.int32),
        ],
    )(x_flat, Wc8, bc8)


def _scatter_body(xwt, dest, att, z2, z1, feat_out, att_out,
                  acc, acc1, zv2, zv1, idxb, rowb, attb, dumpb, dump1):
    core = lax.axis_index("c")
    sid = lax.axis_index("s")

    pltpu.sync_copy(z2, zv2)
    pltpu.sync_copy(z1, zv1)

    def run_task(b, ch, is_att):
        if is_att:
            pltpu.sync_copy(zv1, acc1.at[pl.ds(sid * PPT, PPT)])
        else:
            for m in range(PPT // ZROWS):
                pltpu.sync_copy(
                    zv2, acc.at[pl.ds(sid * PPT + m * ZROWS, ZROWS)])
        plsc.subcore_barrier()

        for k in range(KMAX):
            g = sid + k * NSUB

            @pl.when(g < NCHUNKS)
            def _():
                pltpu.sync_copy(dest.at[pl.ds(b * HW + g * CHUNK, CHUNK)],
                                idxb.at[0])
                if is_att:
                    pltpu.sync_copy(att.at[pl.ds(b * HW + g * CHUNK, CHUNK)],
                                    attb.at[0])
                    pltpu.sync_copy(attb.at[0], acc1.at[idxb.at[0]],
                                    add=True)
                else:
                    pltpu.sync_copy(
                        xwt.at[b * NCH + ch, pl.ds(g * CHUNK, CHUNK)], rowb)
                    pltpu.sync_copy(rowb, acc.at[idxb.at[0]], add=True)

        plsc.subcore_barrier()

        if is_att:
            pltpu.sync_copy(acc1.at[pl.ds(sid * PPT, PPT)], dump1)
            pltpu.sync_copy(dump1,
                            att_out.at[pl.ds(b * HW + sid * PPT, PPT)])
        else:
            for m in range(PPT // ZROWS):
                off = sid * PPT + m * ZROWS
                pltpu.sync_copy(acc.at[pl.ds(off, ZROWS)], dumpb)
                pltpu.sync_copy(dumpb,
                                feat_out.at[b * NCH + ch, pl.ds(off, ZROWS)])
        plsc.subcore_barrier()

    for t in range(8):
        b = core * 2 + t // 4
        ch = t % 4
        run_task(b, ch % NCH, ch == NCH)


def _scatter(xw_t, dest, att):
    z2 = jnp.zeros((ZROWS, CCH), jnp.float32)
    z1 = jnp.zeros((PPT,), jnp.float32)
    mesh = plsc.VectorSubcoreMesh(core_axis_name="c", subcore_axis_name="s")
    kern = pl.kernel(
        _scatter_body,
        mesh=mesh,
        out_type=[
            jax.ShapeDtypeStruct((B * NCH, HW, CCH), jnp.float32),
            jax.ShapeDtypeStruct((B * HW,), jnp.float32),
        ],
        scratch_types=[
            pltpu.VMEM_SHARED((HW, CCH), jnp.float32),
            pltpu.VMEM_SHARED((HW,), jnp.float32),
            pltpu.VMEM((ZROWS, CCH), jnp.float32),
            pltpu.VMEM((PPT,), jnp.float32),
            pltpu.VMEM((1, CHUNK), jnp.int32),
            pltpu.VMEM((CHUNK, CCH), jnp.float32),
            pltpu.VMEM((1, CHUNK), jnp.float32),
            pltpu.VMEM((ZROWS, CCH), jnp.float32),
            pltpu.VMEM((PPT,), jnp.float32),
        ],
        compiler_params=pltpu.CompilerParams(use_tc_tiling_on_sc=False),
    )
    return kern(xw_t, dest, att, z2, z1)


def _final_body(feat_ref, att_ref, out_ref):
    asum = att_ref[0, 0]
    r = (1.0 / (asum + EPS))[None, :]
    for c3 in range(NCH):
        out_ref[0, pl.ds(c3 * CCH, CCH)] = feat_ref[c3].T * r


def _finalize(featacc, attacc):
    return pl.pallas_call(
        _final_body,
        grid=(B, NBLK),
        in_specs=[
            pl.BlockSpec((NCH, BLK, CCH), lambda b, j: (b, j, 0)),
            pl.BlockSpec((1, 1, BLK), lambda b, j: (b, 0, j)),
        ],
        out_specs=pl.BlockSpec((1, C, BLK), lambda b, j: (b, 0, j)),
        out_shape=jax.ShapeDtypeStruct((B, C, HW), jnp.float32),
    )(featacc, attacc)


def kernel(x, Wc, bc):
    x_flat = x.reshape(B, C, HW)
    Wc8 = jnp.zeros((8, C), jnp.float32).at[:3].set(Wc)
    bc8 = jnp.zeros((8, 1), jnp.float32).at[:3, 0].set(bc)

    xw_t, att3, dest3, offset, destination = _project(x_flat, Wc8, bc8)
    dest = dest3.reshape(B * HW)
    att = att3.reshape(B * HW)
    featacc, attacc = _scatter(xw_t, dest, att)
    out = _finalize(featacc, attacc.reshape(B, 1, HW))

    return (out.reshape(B, C, H, W),
            offset.reshape(B, 2, H, W),
            destination.reshape(B, 2, H, W))


# speedup vs baseline: 1.3007x; 1.3007x over previous
import jax
import jax.numpy as jnp
from jax import lax
from jax.experimental import pallas as pl
from jax.experimental.pallas import tpu as pltpu
from jax.experimental.pallas import tpu_sc as plsc

EPS = 1e-05

B, C, H, W = 4, 96, 224, 224
HW = H * W
BLK = 3584
NBLK = HW // BLK

NSUB = 16
CCH = 32
NCH = C // CCH
PPT = HW // NSUB
CHUNK = 128
NCHUNKS = HW // CHUNK
KMAX = (NCHUNKS + NSUB - 1) // NSUB
ZROWS = 196
HW4 = HW // 4


def _proj_body(x_ref, w_ref, b_ref, xwt_ref, att_ref, dest_ref, off_ref,
               dst_ref):
    j = pl.program_id(1)
    xblk = x_ref[0]
    oa = jnp.dot(w_ref[...], xblk, preferred_element_type=jnp.float32)
    oa = oa + b_ref[...]
    off_y = oa[0:1] * float(H)
    off_x = oa[1:2] * float(W)
    att = jnp.exp(oa[2:3])

    p = j * BLK + lax.broadcasted_iota(jnp.int32, (1, BLK), 1)
    gy = (p // W).astype(jnp.float32)
    gx = (p - (p // W) * W).astype(jnp.float32)
    dy = jnp.round(gy + off_y).astype(jnp.int32)
    dx = jnp.round(gx + off_x).astype(jnp.int32)
    cy = jnp.clip(dy, 0, H - 1)
    cx = jnp.clip(dx, 0, W - 1)
    dest_ref[0, 0] = (cy * W + cx)[0]

    off_ref[0, 0] = off_y[0]
    off_ref[0, 1] = off_x[0]
    dst_ref[0, 0] = dy[0]
    dst_ref[0, 1] = dx[0]
    att_ref[0, 0] = att[0]
    for c3 in range(NCH):
        xwt_ref[c3] = (xblk[c3 * CCH:(c3 + 1) * CCH] * att).T


def _project(x_flat, Wc8, bc8):
    return pl.pallas_call(
        _proj_body,
        grid=(B, NBLK),
        in_specs=[
            pl.BlockSpec((1, C, BLK), lambda b, j: (b, 0, j)),
            pl.BlockSpec((8, C), lambda b, j: (0, 0)),
            pl.BlockSpec((8, 1), lambda b, j: (0, 0)),
        ],
        out_specs=[
            pl.BlockSpec((NCH, BLK, CCH), lambda b, j: (b, j, 0)),
            pl.BlockSpec((1, 1, BLK), lambda b, j: (b, 0, j)),
            pl.BlockSpec((1, 1, BLK), lambda b, j: (b, 0, j)),
            pl.BlockSpec((1, 2, BLK), lambda b, j: (b, 0, j)),
            pl.BlockSpec((1, 2, BLK), lambda b, j: (b, 0, j)),
        ],
        out_shape=[
            jax.ShapeDtypeStruct((B * NCH, HW, CCH), jnp.float32),
            jax.ShapeDtypeStruct((B, 1, HW), jnp.float32),
            jax.ShapeDtypeStruct((B, 1, HW), jnp.int32),
            jax.ShapeDtypeStruct((B, 2, HW), jnp.float32),
            jax.ShapeDtypeStruct((B, 2, HW), jnp.int32),
        ],
    )(x_flat, Wc8, bc8)


def _scatter_body(xwt, dest, att, z2, z1, feat_out, att_out,
                  acc, acc1, zv2, zv1, idxb, rowb, attb, dumpb, dump1):
    core = lax.axis_index("c")
    sid = lax.axis_index("s")

    pltpu.sync_copy(z2, zv2)
    pltpu.sync_copy(z1, zv1)

    def run_task(b, ch, is_att):
        if is_att:
            pltpu.sync_copy(zv1, acc1.at[pl.ds(sid * PPT, PPT)])
        else:
            for m in range(PPT // ZROWS):
                pltpu.sync_copy(
                    zv2, acc.at[pl.ds(sid * PPT + m * ZROWS, ZROWS)])
        plsc.subcore_barrier()

        for k in range(KMAX):
            g = sid + k * NSUB

            @pl.when(g < NCHUNKS)
            def _():
                pltpu.sync_copy(dest.at[pl.ds(b * HW + g * CHUNK, CHUNK)],
                                idxb.at[0])
                if is_att:
                    pltpu.sync_copy(att.at[pl.ds(b * HW + g * CHUNK, CHUNK)],
                                    attb.at[0])
                    pltpu.sync_copy(attb.at[0], acc1.at[idxb.at[0]],
                                    add=True)
                else:
                    pltpu.sync_copy(
                        xwt.at[b * NCH + ch, pl.ds(g * CHUNK, CHUNK)], rowb)
                    pltpu.sync_copy(rowb, acc.at[idxb.at[0]], add=True)

        plsc.subcore_barrier()

        if is_att:
            pltpu.sync_copy(acc1.at[pl.ds(sid * PPT, PPT)], dump1)
            pltpu.sync_copy(dump1,
                            att_out.at[pl.ds(b * HW + sid * PPT, PPT)])
        else:
            for m in range(PPT // ZROWS):
                off = sid * PPT + m * ZROWS
                pltpu.sync_copy(acc.at[pl.ds(off, ZROWS)], dumpb)
                pltpu.sync_copy(dumpb,
                                feat_out.at[b * NCH + ch, pl.ds(off, ZROWS)])
        plsc.subcore_barrier()

    for t in range(8):
        b = core * 2 + t // 4
        ch = t % 4
        run_task(b, ch % NCH, ch == NCH)


def _scatter(xw_t, dest, att):
    z2 = jnp.zeros((ZROWS, CCH), jnp.float32)
    z1 = jnp.zeros((PPT,), jnp.float32)
    mesh = plsc.VectorSubcoreMesh(core_axis_name="c", subcore_axis_name="s")
    kern = pl.kernel(
        _scatter_body,
        mesh=mesh,
        out_type=[
            jax.ShapeDtypeStruct((B * NCH, HW, CCH), jnp.float32),
            jax.ShapeDtypeStruct((B * HW,), jnp.float32),
        ],
        scratch_types=[
            pltpu.VMEM_SHARED((HW, CCH), jnp.float32),
            pltpu.VMEM_SHARED((HW,), jnp.float32),
            pltpu.VMEM((ZROWS, CCH), jnp.float32),
            pltpu.VMEM((PPT,), jnp.float32),
            pltpu.VMEM((1, CHUNK), jnp.int32),
            pltpu.VMEM((CHUNK, CCH), jnp.float32),
            pltpu.VMEM((1, CHUNK), jnp.float32),
            pltpu.VMEM((ZROWS, CCH), jnp.float32),
            pltpu.VMEM((PPT,), jnp.float32),
        ],
        compiler_params=pltpu.CompilerParams(use_tc_tiling_on_sc=False),
    )
    return kern(xw_t, dest, att, z2, z1)


def _final_body(feat_ref, att_ref, out_ref):
    asum = att_ref[0, 0]
    r = (1.0 / (asum + EPS))[None, :]
    for c3 in range(NCH):
        out_ref[0, pl.ds(c3 * CCH, CCH)] = feat_ref[c3].T * r


def _finalize(featacc, attacc):
    return pl.pallas_call(
        _final_body,
        grid=(B, NBLK),
        in_specs=[
            pl.BlockSpec((NCH, BLK, CCH), lambda b, j: (b, j, 0)),
            pl.BlockSpec((1, 1, BLK), lambda b, j: (b, 0, j)),
        ],
        out_specs=pl.BlockSpec((1, C, BLK), lambda b, j: (b, 0, j)),
        out_shape=jax.ShapeDtypeStruct((B, C, HW), jnp.float32),
    )(featacc, attacc)


def kernel(x, Wc, bc):
    x_flat = x.reshape(B, C, HW)
    Wc8 = jnp.zeros((8, C), jnp.float32).at[:3].set(Wc)
    bc8 = jnp.zeros((8, 1), jnp.float32).at[:3, 0].set(bc)

    xw_t, att3, dest3, offset, destination = _project(x_flat, Wc8, bc8)
    dest = dest3.reshape(B * HW)
    att = att3.reshape(B * HW)
    featacc, attacc = _scatter(xw_t, dest, att)
    out = _finalize(featacc, attacc.reshape(B, 1, HW))

    return (out.reshape(B, C, H, W),
            offset.reshape(B, 2, H, W),
            destination.reshape(B, 2, H, W))
